# baseline (speedup 1.0000x reference)
import functools

import jax
import jax.numpy as jnp
from jax import lax
from jax.experimental import pallas as pl
from jax.experimental.pallas import tpu as pltpu
from jax.experimental.pallas import tpu_sc as plsc

_NC, _NS = 2, 16
_NW = _NC * _NS
_B = 4096
_D = 64
_N = 100000
_FPW = 4 * _D // _NW
_EPS = 1e-5
_L = 16


_H = 49920
_H1 = _N - _H


def _sc_gather_body(uid_hbm, iid_hbm, gu_t, mu_t, gi_t, mi_t,
                    out_hbm, idx_u, idx_i, buf0, buf1, outa, outb,
                    sem0, sem1, semo0, semo1):
    wid = lax.axis_index("s") * _NC + lax.axis_index("c")
    _FPT = _D // _NW
    fbase = wid * _FPT
    pltpu.sync_copy(uid_hbm, idx_u)
    pltpu.sync_copy(iid_hbm, idx_i)

    tabs = ((gu_t, idx_u), (mu_t, idx_u), (gi_t, idx_i), (mi_t, idx_i))
    tasks = [(t, f, h) for t in range(4) for f in range(_FPT)
             for h in range(2)]
    bufs = (buf0, buf1)

    sems = (sem0, sem1)

    def fire(k):
        t, f, h = tasks[k]
        src = tabs[t][0].at[fbase + f,
                            pl.ds(h * _H, _H if h == 0 else _H1)]
        dst = bufs[k % 2].at[pl.ds(0, _H if h == 0 else _H1)]
        return pltpu.async_copy(src, dst, sems[k % 2])

    outs = (outa, outb)
    out_pending = [None, None]
    pending = fire(0)
    for k, (t, f, h) in enumerate(tasks):
        nxt = fire(k + 1) if k + 1 < len(tasks) else None
        pending.wait()
        pending = nxt
        idx_v = tabs[t][1]
        buf = bufs[k % 2]
        par = (t * _FPT + f) % 2
        out_v = outs[par]
        if h == 0 and out_pending[par] is not None:
            out_pending[par].wait()
            out_pending[par] = None

        _U = 1
        if h == 0:
            def grp(g, c, idx_v=idx_v, buf=buf, out_v=out_v):
                sls = [pl.ds((g * _U + k) * _L, _L) for k in range(_U)]
                ivs = [idx_v[sl] for sl in sls]
                for k in range(_U):
                    m = ivs[k] < _H
                    out_v[sls[k]] = plsc.load_gather(buf, [ivs[k]], mask=m)
                return c
        else:
            def grp(g, c, idx_v=idx_v, buf=buf, out_v=out_v):
                sls = [pl.ds((g * _U + k) * _L, _L) for k in range(_U)]
                ivs = [idx_v[sl] for sl in sls]
                for k in range(_U):
                    m = ivs[k] >= _H
                    vals = plsc.load_gather(buf, [ivs[k] - _H], mask=m)
                    out_v[sls[k]] = jnp.where(m, vals, out_v[sls[k]])
                return c

        lax.fori_loop(0, _B // (_U * _L), grp, 0)
        if h == 1:
            out_pending[par] = pltpu.async_copy(
                out_v, out_hbm.at[t * _D + fbase + f],
                (semo0, semo1)[par])
    for d in out_pending:
        if d is not None:
            d.wait()


@jax.jit
def _sc_gather(user_ids, item_ids, gu_t, mu_t, gi_t, mi_t):
    mesh = plsc.VectorSubcoreMesh(core_axis_name="c", subcore_axis_name="s")
    f = pl.kernel(
        _sc_gather_body,
        out_type=jax.ShapeDtypeStruct((4 * _D, _B), jnp.float32),
        mesh=mesh,
        compiler_params=pltpu.CompilerParams(needs_layout_passes=False),
        scratch_types=[
            pltpu.VMEM((_B,), jnp.int32),
            pltpu.VMEM((_B,), jnp.int32),
            pltpu.VMEM((_H1,), jnp.float32),
            pltpu.VMEM((_H1,), jnp.float32),
            pltpu.VMEM((_B,), jnp.float32),
            pltpu.VMEM((_B,), jnp.float32),
            pltpu.SemaphoreType.DMA,
            pltpu.SemaphoreType.DMA,
            pltpu.SemaphoreType.DMA,
            pltpu.SemaphoreType.DMA,
        ],
    )
    return f(user_ids, item_ids, gu_t, mu_t, gi_t, mi_t)


def _mlp_body(gu_ref, mu_ref, gi_ref, mi_ref,
              w1_ref, w2_ref, w3_ref,
              b1_ref, s1_ref, be1_ref, b2_ref, s2_ref, be2_ref,
              b3_ref, s3_ref, be3_ref, wog_ref, woh_ref, bo_ref,
              out_ref):
    cdim = (((1,), (0,)), ((), ()))
    w1 = w1_ref[...]
    h = lax.dot_general(w1[:, :_D], mu_ref[...], cdim,
                        preferred_element_type=jnp.float32)
    h = h + lax.dot_general(w1[:, _D:], mi_ref[...], cdim,
                            preferred_element_type=jnp.float32)
    h = jnp.maximum(h + b1_ref[...], 0.0) * s1_ref[...] + be1_ref[...]
    h = lax.dot_general(w2_ref[...], h, cdim,
                        preferred_element_type=jnp.float32)
    h = jnp.maximum(h + b2_ref[...], 0.0) * s2_ref[...] + be2_ref[...]
    h = lax.dot_general(w3_ref[...], h, cdim,
                        preferred_element_type=jnp.float32)
    h = jnp.maximum(h + b3_ref[...], 0.0) * s3_ref[...] + be3_ref[...]
    logit = (jnp.sum((gu_ref[...] * gi_ref[...]) * wog_ref[...], axis=0)
             + jnp.sum(h * woh_ref[...], axis=0) + bo_ref[0])
    out_ref[...] = jax.nn.sigmoid(logit)


@jax.jit
def _mlp_tower(bands, w1, w2, w3,
               b1, s1, be1, b2, s2, be2, b3, s3, be3, wog, woh, bo):
    nblk = 4
    cols = _B // nblk
    full = lambda i: (0, 0)
    band = lambda r: pl.BlockSpec((_D, cols), lambda i, r=r: (r, i))
    return pl.pallas_call(
        _mlp_body,
        grid=(nblk,),
        in_specs=[
            band(0), band(1), band(2), band(3),
            pl.BlockSpec((256, 128), full),
            pl.BlockSpec((128, 256), full), pl.BlockSpec((_D, 128), full),
            pl.BlockSpec((256, 1), full), pl.BlockSpec((256, 1), full),
            pl.BlockSpec((256, 1), full),
            pl.BlockSpec((128, 1), full), pl.BlockSpec((128, 1), full),
            pl.BlockSpec((128, 1), full),
            pl.BlockSpec((_D, 1), full), pl.BlockSpec((_D, 1), full),
            pl.BlockSpec((_D, 1), full),
            pl.BlockSpec((_D, 1), full), pl.BlockSpec((_D, 1), full),
            pl.BlockSpec(memory_space=pltpu.SMEM),
        ],
        out_specs=pl.BlockSpec((cols,), lambda i: (i,)),
        out_shape=jax.ShapeDtypeStruct((_B,), jnp.float32),
    )(bands, bands, bands, bands, w1, w2, w3,
      b1, s1, be1, b2, s2, be2, b3, s3, be3, wog, woh, bo)


def kernel(user_ids, item_ids, gmf_user_tab, gmf_item_tab, mlp_user_tab,
           mlp_item_tab, W1, b1, g1, be1, W2, b2, g2, be2, W3, b3, g3, be3,
           Wo, bo):
    user_ids = user_ids.astype(jnp.int32)
    item_ids = item_ids.astype(jnp.int32)
    bands = _sc_gather(user_ids, item_ids,
                       gmf_user_tab.T, mlp_user_tab.T,
                       gmf_item_tab.T, mlp_item_tab.T)
    inv = 1.0 / jnp.sqrt(1.0 + _EPS)
    col = lambda v: v.reshape(-1, 1)
    return _mlp_tower(
        bands, W1, W2, W3,
        col(b1), col(inv * g1), col(be1),
        col(b2), col(inv * g2), col(be2),
        col(b3), col(inv * g3), col(be3),
        col(Wo[0, :_D]), col(Wo[0, _D:]), bo)


# speedup vs baseline: 1.0309x; 1.0309x over previous
import functools

import jax
import jax.numpy as jnp
from jax import lax
from jax.experimental import pallas as pl
from jax.experimental.pallas import tpu as pltpu
from jax.experimental.pallas import tpu_sc as plsc

_NC, _NS = 2, 16
_NW = _NC * _NS
_B = 4096
_D = 64
_N = 100000
_FPW = 4 * _D // _NW
_EPS = 1e-5
_L = 16


_H = 49920
_H1 = _N - _H


def _sc_gather_body(uid_hbm, iid_hbm, gu_t, mu_t, gi_t, mi_t,
                    out_hbm, idx_u, idx_i, buf0, buf1, out_v, sem0, sem1):
    wid = lax.axis_index("s") * _NC + lax.axis_index("c")
    _FPT = _D // _NW
    fbase = wid * _FPT
    pltpu.sync_copy(uid_hbm, idx_u)
    pltpu.sync_copy(iid_hbm, idx_i)

    tabs = ((gu_t, idx_u), (mu_t, idx_u), (gi_t, idx_i), (mi_t, idx_i))
    tasks = [(t, f, h) for t in range(4) for f in range(_FPT)
             for h in range(2)]
    bufs = (buf0, buf1)

    sems = (sem0, sem1)

    def fire(k):
        t, f, h = tasks[k]
        src = tabs[t][0].at[fbase + f,
                            pl.ds(h * _H, _H if h == 0 else _H1)]
        dst = bufs[k % 2].at[pl.ds(0, _H if h == 0 else _H1)]
        return pltpu.async_copy(src, dst, sems[k % 2])

    pending = fire(0)
    for k, (t, f, h) in enumerate(tasks):
        nxt = fire(k + 1) if k + 1 < len(tasks) else None
        pending.wait()
        pending = nxt
        idx_v = tabs[t][1]
        buf = bufs[k % 2]

        _U = 1
        if h == 0:
            def grp(g, c, idx_v=idx_v, buf=buf, out_v=out_v):
                sls = [pl.ds((g * _U + k) * _L, _L) for k in range(_U)]
                ivs = [idx_v[sl] for sl in sls]
                for k in range(_U):
                    m = ivs[k] < _H
                    out_v[sls[k]] = plsc.load_gather(buf, [ivs[k]], mask=m)
                return c
        else:
            def grp(g, c, idx_v=idx_v, buf=buf, out_v=out_v):
                sls = [pl.ds((g * _U + k) * _L, _L) for k in range(_U)]
                ivs = [idx_v[sl] for sl in sls]
                for k in range(_U):
                    m = ivs[k] >= _H
                    vals = plsc.load_gather(buf, [ivs[k] - _H], mask=m)
                    out_v[sls[k]] = jnp.where(m, vals, out_v[sls[k]])
                return c

        lax.fori_loop(0, _B // (_U * _L), grp, 0)
        if h == 1:
            pltpu.sync_copy(out_v, out_hbm.at[t * _D + fbase + f])


@jax.jit
def _sc_gather(user_ids, item_ids, gu_t, mu_t, gi_t, mi_t):
    mesh = plsc.VectorSubcoreMesh(core_axis_name="c", subcore_axis_name="s")
    f = pl.kernel(
        _sc_gather_body,
        out_type=jax.ShapeDtypeStruct((4 * _D, _B), jnp.float32),
        mesh=mesh,
        compiler_params=pltpu.CompilerParams(needs_layout_passes=False),
        scratch_types=[
            pltpu.VMEM((_B,), jnp.int32),
            pltpu.VMEM((_B,), jnp.int32),
            pltpu.VMEM((_H1,), jnp.float32),
            pltpu.VMEM((_H1,), jnp.float32),
            pltpu.VMEM((_B,), jnp.float32),
            pltpu.SemaphoreType.DMA,
            pltpu.SemaphoreType.DMA,
        ],
    )
    return f(user_ids, item_ids, gu_t, mu_t, gi_t, mi_t)


def _mlp_body(gu_ref, mu_ref, gi_ref, mi_ref,
              w1_ref, w2_ref, w3_ref,
              b1_ref, s1_ref, be1_ref, b2_ref, s2_ref, be2_ref,
              b3_ref, s3_ref, be3_ref, wog_ref, woh_ref, bo_ref,
              out_ref):
    cdim = (((1,), (0,)), ((), ()))
    w1 = w1_ref[...]
    h = lax.dot_general(w1[:, :_D], mu_ref[...], cdim,
                        preferred_element_type=jnp.float32)
    h = h + lax.dot_general(w1[:, _D:], mi_ref[...], cdim,
                            preferred_element_type=jnp.float32)
    h = jnp.maximum(h + b1_ref[...], 0.0) * s1_ref[...] + be1_ref[...]
    h = lax.dot_general(w2_ref[...], h, cdim,
                        preferred_element_type=jnp.float32)
    h = jnp.maximum(h + b2_ref[...], 0.0) * s2_ref[...] + be2_ref[...]
    h = lax.dot_general(w3_ref[...], h, cdim,
                        preferred_element_type=jnp.float32)
    h = jnp.maximum(h + b3_ref[...], 0.0) * s3_ref[...] + be3_ref[...]
    logit = (jnp.sum((gu_ref[...] * gi_ref[...]) * wog_ref[...], axis=0)
             + jnp.sum(h * woh_ref[...], axis=0) + bo_ref[0])
    out_ref[...] = jax.nn.sigmoid(logit)


@jax.jit
def _mlp_tower(bands, w1, w2, w3,
               b1, s1, be1, b2, s2, be2, b3, s3, be3, wog, woh, bo):
    nblk = 4
    cols = _B // nblk
    full = lambda i: (0, 0)
    band = lambda r: pl.BlockSpec((_D, cols), lambda i, r=r: (r, i))
    return pl.pallas_call(
        _mlp_body,
        grid=(nblk,),
        in_specs=[
            band(0), band(1), band(2), band(3),
            pl.BlockSpec((256, 128), full),
            pl.BlockSpec((128, 256), full), pl.BlockSpec((_D, 128), full),
            pl.BlockSpec((256, 1), full), pl.BlockSpec((256, 1), full),
            pl.BlockSpec((256, 1), full),
            pl.BlockSpec((128, 1), full), pl.BlockSpec((128, 1), full),
            pl.BlockSpec((128, 1), full),
            pl.BlockSpec((_D, 1), full), pl.BlockSpec((_D, 1), full),
            pl.BlockSpec((_D, 1), full),
            pl.BlockSpec((_D, 1), full), pl.BlockSpec((_D, 1), full),
            pl.BlockSpec(memory_space=pltpu.SMEM),
        ],
        out_specs=pl.BlockSpec((cols,), lambda i: (i,)),
        out_shape=jax.ShapeDtypeStruct((_B,), jnp.float32),
    )(bands, bands, bands, bands, w1, w2, w3,
      b1, s1, be1, b2, s2, be2, b3, s3, be3, wog, woh, bo)


def kernel(user_ids, item_ids, gmf_user_tab, gmf_item_tab, mlp_user_tab,
           mlp_item_tab, W1, b1, g1, be1, W2, b2, g2, be2, W3, b3, g3, be3,
           Wo, bo):
    user_ids = user_ids.astype(jnp.int32)
    item_ids = item_ids.astype(jnp.int32)
    bands = _sc_gather(user_ids, item_ids,
                       gmf_user_tab.T, mlp_user_tab.T,
                       gmf_item_tab.T, mlp_item_tab.T)
    inv = 1.0 / jnp.sqrt(1.0 + _EPS)
    col = lambda v: v.reshape(-1, 1)
    return _mlp_tower(
        bands, W1, W2, W3,
        col(b1), col(inv * g1), col(be1),
        col(b2), col(inv * g2), col(be2),
        col(b3), col(inv * g3), col(be3),
        col(Wo[0, :_D]), col(Wo[0, _D:]), bo)
